# baseline (speedup 1.0000x reference)
import functools

import jax
import jax.numpy as jnp
from jax.experimental import pallas as pl
from jax.experimental.pallas import tpu as pltpu

_M = 1024
_D = 256
_BLK = 512


def _vq_kernel(nblocks, n_rows, x_ref, embt_ref, embt_bf_ref, emb_bf_ref,
               q_ref, inds_ref, cp_ref, pp_ref,
               e2_ref, psum_ref, hist_ref, riota_ref):
    i = pl.program_id(0)

    @pl.when(i == 0)
    def _init():
        embt = embt_ref[...]
        e2_ref[...] = jnp.sum(embt * embt, axis=0, keepdims=True)
        psum_ref[...] = jnp.zeros_like(psum_ref)
        hist_ref[...] = jnp.zeros_like(hist_ref)
        iota_i = jax.lax.broadcasted_iota(jnp.int32, riota_ref.shape, 1)
        riota_ref[...] = (_M - iota_i).astype(jnp.float32)

    x = x_ref[...]
    x2 = jnp.sum(x * x, axis=1, keepdims=True)
    s = jnp.dot(x.astype(jnp.bfloat16), embt_bf_ref[...],
                preferred_element_type=jnp.float32)
    dmap = 2.0 * s - (e2_ref[...] + x2)

    m = jnp.max(dmap, axis=1, keepdims=True)
    masked = jnp.where(dmap == m, riota_ref[...], 0.0)
    r = jnp.max(masked, axis=1, keepdims=True)
    k = (float(_M) - r).astype(jnp.int32)
    inds_ref[...] = k

    p = jnp.exp(dmap - m)
    w = (1.0 / jnp.sum(p, axis=1, keepdims=True)).astype(jnp.bfloat16)
    psum_ref[...] += jax.lax.dot_general(
        w, p.astype(jnp.bfloat16), (((0,), (0,)), ((), ())),
        preferred_element_type=jnp.float32)

    oh = jnp.where(masked == r, 1.0, 0.0).astype(jnp.bfloat16)
    ones_col = jnp.full((x.shape[0], 1), jnp.bfloat16(1.0), jnp.bfloat16)
    hist_ref[...] += jax.lax.dot_general(
        ones_col, oh, (((0,), (0,)), ((), ())),
        preferred_element_type=jnp.float32)

    q_ref[...] = jnp.dot(oh, emb_bf_ref[...], preferred_element_type=jnp.float32)

    @pl.when(i == nblocks - 1)
    def _finish():
        inv_n = 1.0 / n_rows
        hp = hist_ref[...] * inv_n
        cp_ref[...] = -jnp.sum(hp * (jnp.log2(hp + 1e-10)), axis=1, keepdims=True)
        ap = psum_ref[...] * inv_n
        pp_ref[...] = -jnp.sum(ap * (jnp.log2(ap + 1e-10)), axis=1, keepdims=True)


def kernel(x, embedding):
    bsz, tsz, csz = x.shape
    n = bsz * tsz
    x_flat = x.reshape(n, csz)
    emb = embedding[0]
    embt = emb.T
    nblocks = n // _BLK

    q, inds, cp, pp = pl.pallas_call(
        functools.partial(_vq_kernel, nblocks, float(n)),
        grid=(nblocks,),
        in_specs=[
            pl.BlockSpec((_BLK, _D), lambda i: (i, 0)),
            pl.BlockSpec((_D, _M), lambda i: (0, 0)),
            pl.BlockSpec((_D, _M), lambda i: (0, 0)),
            pl.BlockSpec((_M, _D), lambda i: (0, 0)),
        ],
        out_specs=[
            pl.BlockSpec((_BLK, _D), lambda i: (i, 0)),
            pl.BlockSpec((_BLK, 1), lambda i: (i, 0)),
            pl.BlockSpec((1, 1), lambda i: (0, 0)),
            pl.BlockSpec((1, 1), lambda i: (0, 0)),
        ],
        out_shape=[
            jax.ShapeDtypeStruct((n, _D), jnp.float32),
            jax.ShapeDtypeStruct((n, 1), jnp.int32),
            jax.ShapeDtypeStruct((1, 1), jnp.float32),
            jax.ShapeDtypeStruct((1, 1), jnp.float32),
        ],
        scratch_shapes=[
            pltpu.VMEM((1, _M), jnp.float32),
            pltpu.VMEM((1, _M), jnp.float32),
            pltpu.VMEM((1, _M), jnp.float32),
            pltpu.VMEM((_BLK, _M), jnp.float32),
        ],
    )(x_flat, embt, embt.astype(jnp.bfloat16), emb.astype(jnp.bfloat16))

    quantized = q.reshape(bsz, tsz, csz)
    quantization_inds = inds.reshape(bsz, tsz, 1)
    return (quantized, cp[0, 0], pp[0, 0], quantization_inds)
